# Initial kernel scaffold; baseline (speedup 1.0000x reference)
import jax
import jax.numpy as jnp
from jax.experimental import pallas as pl


def kernel(x, modality_type, n1_s, n1_b, n2_s, n2_b, attn_in_w, attn_in_b, attn_out_w, attn_out_b, router_w, router_b, spec_ln_s, spec_ln_b, spec_w1, spec_b1, spec_w2, spec_b2, gen_ln_s, gen_ln_b, gen_w1, gen_b1, gen_w2, gen_b2):
    raise NotImplementedError("write your pallas kernel here")


# speedup vs baseline: 1.7653x; 1.7653x over previous
import functools

import jax
import jax.numpy as jnp
from jax.experimental import pallas as pl

S, D, H, E, DFF = 2048, 1024, 16, 8, 4096
DH = D // H
NEG = -1e30


def _ln_qkv_body(x_ref, s_ref, b_ref, w_ref, wb_ref, o_ref):
    xb = x_ref[...]
    m = jnp.mean(xb, axis=-1, keepdims=True)
    v = jnp.mean((xb - m) ** 2, axis=-1, keepdims=True)
    xn = (xb - m) * jax.lax.rsqrt(v + 1e-5) * s_ref[...] + b_ref[...]
    o_ref[...] = jax.lax.dot_general(
        xn, w_ref[...], (((1,), (1,)), ((), ())),
        preferred_element_type=jnp.float32) + wb_ref[...]


def _ln_qkv(x2, n1_s, n1_b, w, b):
    BM, BN = 256, 512
    return pl.pallas_call(
        _ln_qkv_body,
        grid=(S // BM, 3 * D // BN),
        in_specs=[
            pl.BlockSpec((BM, D), lambda i, j: (i, 0)),
            pl.BlockSpec((1, D), lambda i, j: (0, 0)),
            pl.BlockSpec((1, D), lambda i, j: (0, 0)),
            pl.BlockSpec((BN, D), lambda i, j: (j, 0)),
            pl.BlockSpec((1, BN), lambda i, j: (0, j)),
        ],
        out_specs=pl.BlockSpec((BM, BN), lambda i, j: (i, j)),
        out_shape=jax.ShapeDtypeStruct((S, 3 * D), jnp.float32),
    )(x2, n1_s.reshape(1, D), n1_b.reshape(1, D), w, b.reshape(1, 3 * D))


def _attn_body(q_ref, k_ref, v_ref, ao_ref, aw_ref):
    h = pl.program_id(1)
    q = q_ref[0]
    k = k_ref[0]
    v = v_ref[0]
    s = jax.lax.dot_general(q, k, (((1,), (1,)), ((), ())),
                            preferred_element_type=jnp.float32) * (1.0 / 8.0)
    mx = jnp.max(s, axis=-1, keepdims=True)
    p = jnp.exp(s - mx)
    p = p / jnp.sum(p, axis=-1, keepdims=True)
    ao_ref[0] = jax.lax.dot_general(p, v, (((1,), (0,)), ((), ())),
                                    preferred_element_type=jnp.float32)

    @pl.when(h == 0)
    def _():
        aw_ref[...] = p * (1.0 / H)

    @pl.when(h != 0)
    def _():
        aw_ref[...] += p * (1.0 / H)


def _attention(qkv):
    qkv3 = qkv.reshape(S, 3 * H, DH).transpose(1, 0, 2)
    BM = 512
    ao3, aw = pl.pallas_call(
        _attn_body,
        grid=(S // BM, H),
        in_specs=[
            pl.BlockSpec((1, BM, DH), lambda i, h: (h, i, 0)),
            pl.BlockSpec((1, S, DH), lambda i, h: (H + h, 0, 0)),
            pl.BlockSpec((1, S, DH), lambda i, h: (2 * H + h, 0, 0)),
        ],
        out_specs=[
            pl.BlockSpec((1, BM, DH), lambda i, h: (h, i, 0)),
            pl.BlockSpec((BM, S), lambda i, h: (i, 0)),
        ],
        out_shape=[
            jax.ShapeDtypeStruct((H, S, DH), jnp.float32),
            jax.ShapeDtypeStruct((S, S), jnp.float32),
        ],
    )(qkv3, qkv3, qkv3)
    return ao3.transpose(1, 0, 2).reshape(S, D), aw


def _post_attn_body(ao_ref, wo_ref, bo_ref, xr_ref, s2_ref, b2_ref,
                    rw_ref, rb_ref,
                    x1_ref, xhat_ref, logits_ref, probs_ref, coef_ref):
    x1 = jax.lax.dot_general(ao_ref[...], wo_ref[...], (((1,), (1,)), ((), ())),
                             preferred_element_type=jnp.float32)
    x1 = x1 + bo_ref[...] + xr_ref[...]
    x1_ref[...] = x1
    m = jnp.mean(x1, axis=-1, keepdims=True)
    v = jnp.mean((x1 - m) ** 2, axis=-1, keepdims=True)
    xhat = (x1 - m) * jax.lax.rsqrt(v + 1e-5)
    xhat_ref[...] = xhat
    xn2 = xhat * s2_ref[...] + b2_ref[...]
    logits = jax.lax.dot_general(xn2, rw_ref[...], (((1,), (1,)), ((), ())),
                                 preferred_element_type=jnp.float32) + rb_ref[...]
    logits_ref[...] = logits
    mx = jnp.max(logits, axis=-1, keepdims=True)
    ex = jnp.exp(logits - mx)
    probs = ex / jnp.sum(ex, axis=-1, keepdims=True)
    probs_ref[...] = probs
    lane = jax.lax.broadcasted_iota(jnp.int32, probs.shape, 1)
    v1 = jnp.max(probs, axis=-1, keepdims=True)
    i1 = jnp.min(jnp.where(probs == v1, lane, E), axis=-1, keepdims=True)
    m1 = lane == i1
    pm = jnp.where(m1, NEG, probs)
    v2 = jnp.max(pm, axis=-1, keepdims=True)
    i2 = jnp.min(jnp.where(pm == v2, lane, E), axis=-1, keepdims=True)
    m2 = lane == i2
    b = jnp.exp(v2 - v1)
    w1 = 1.0 / (1.0 + b)
    w2 = b / (1.0 + b)
    coef_ref[...] = w1 * m1.astype(jnp.float32) + w2 * m2.astype(jnp.float32)


def _post_attn(ao, wo, bo, xres, n2_s, n2_b, rw, rb):
    BM = 256
    return pl.pallas_call(
        _post_attn_body,
        grid=(S // BM,),
        in_specs=[
            pl.BlockSpec((BM, D), lambda i: (i, 0)),
            pl.BlockSpec((D, D), lambda i: (0, 0)),
            pl.BlockSpec((1, D), lambda i: (0, 0)),
            pl.BlockSpec((BM, D), lambda i: (i, 0)),
            pl.BlockSpec((1, D), lambda i: (0, 0)),
            pl.BlockSpec((1, D), lambda i: (0, 0)),
            pl.BlockSpec((E, D), lambda i: (0, 0)),
            pl.BlockSpec((1, E), lambda i: (0, 0)),
        ],
        out_specs=[
            pl.BlockSpec((BM, D), lambda i: (i, 0)),
            pl.BlockSpec((BM, D), lambda i: (i, 0)),
            pl.BlockSpec((BM, E), lambda i: (i, 0)),
            pl.BlockSpec((BM, E), lambda i: (i, 0)),
            pl.BlockSpec((BM, E), lambda i: (i, 0)),
        ],
        out_shape=[
            jax.ShapeDtypeStruct((S, D), jnp.float32),
            jax.ShapeDtypeStruct((S, D), jnp.float32),
            jax.ShapeDtypeStruct((S, E), jnp.float32),
            jax.ShapeDtypeStruct((S, E), jnp.float32),
            jax.ShapeDtypeStruct((S, E), jnp.float32),
        ],
    )(ao, wo, bo.reshape(1, D), xres, n2_s.reshape(1, D), n2_b.reshape(1, D),
      rw, rb.reshape(1, E))


def _gelu(x):
    return x * 0.5 * (1.0 + jax.lax.erf(x * (2.0 ** -0.5)))


def _experts_body(nE, xh_ref, acc_ref, sc_ref, ls_ref, lb_ref,
                  w1_ref, b1_ref, w2_ref, b2_ref, o_ref):
    e = pl.program_id(1)
    f = pl.program_id(2)

    @pl.when((e == 0) & (f == 0))
    def _():
        o_ref[...] = acc_ref[...]

    xin = xh_ref[...] * ls_ref[0] + lb_ref[0]
    h = _gelu(jax.lax.dot_general(xin, w1_ref[0], (((1,), (1,)), ((), ())),
                                  preferred_element_type=jnp.float32) + b1_ref[0])
    y = jax.lax.dot_general(h, w2_ref[0], (((1,), (1,)), ((), ())),
                            preferred_element_type=jnp.float32)
    lane = jax.lax.broadcasted_iota(jnp.int32, sc_ref.shape, 1)
    scale = jnp.sum(jnp.where(lane == e, sc_ref[...], 0.0), axis=-1,
                    keepdims=True)

    @pl.when(f == 0)
    def _():
        o_ref[...] += y * scale + scale * b2_ref[0]

    @pl.when(f != 0)
    def _():
        o_ref[...] += y * scale


def _experts(xhat, acc, scales, ln_s, ln_b, w1, b1, w2, b2):
    nE = w1.shape[0]
    BM, BF = 256, 1024
    return pl.pallas_call(
        functools.partial(_experts_body, nE),
        grid=(S // BM, nE, DFF // BF),
        in_specs=[
            pl.BlockSpec((BM, D), lambda s, e, f: (s, 0)),
            pl.BlockSpec((BM, D), lambda s, e, f: (s, 0)),
            pl.BlockSpec((BM, nE), lambda s, e, f: (s, 0)),
            pl.BlockSpec((1, 1, D), lambda s, e, f: (e, 0, 0)),
            pl.BlockSpec((1, 1, D), lambda s, e, f: (e, 0, 0)),
            pl.BlockSpec((1, BF, D), lambda s, e, f: (e, f, 0)),
            pl.BlockSpec((1, 1, BF), lambda s, e, f: (e, 0, f)),
            pl.BlockSpec((1, D, BF), lambda s, e, f: (e, 0, f)),
            pl.BlockSpec((1, 1, D), lambda s, e, f: (e, 0, 0)),
        ],
        out_specs=pl.BlockSpec((BM, D), lambda s, e, f: (s, 0)),
        out_shape=jax.ShapeDtypeStruct((S, D), jnp.float32),
    )(xhat, acc, scales, ln_s.reshape(nE, 1, D), ln_b.reshape(nE, 1, D),
      w1, b1.reshape(nE, 1, DFF), w2, b2.reshape(nE, 1, D))


def kernel(x, modality_type, n1_s, n1_b, n2_s, n2_b, attn_in_w, attn_in_b,
           attn_out_w, attn_out_b, router_w, router_b, spec_ln_s, spec_ln_b,
           spec_w1, spec_b1, spec_w2, spec_b2, gen_ln_s, gen_ln_b, gen_w1,
           gen_b1, gen_w2, gen_b2):
    x2 = x[0]
    mflat = modality_type[0]

    qkv = _ln_qkv(x2, n1_s, n1_b, attn_in_w, attn_in_b)
    ao, aw = _attention(qkv)
    x1, xhat, logits, probs, coef = _post_attn(
        ao, attn_out_w, attn_out_b, x2, n2_s, n2_b, router_w, router_b)

    ones = jnp.ones((S,), jnp.float32)
    scales_spec = jnp.stack(
        [ones, (mflat == 0).astype(jnp.float32),
         (mflat == 1).astype(jnp.float32)], axis=1)
    y1 = _experts(xhat, x1, scales_spec, spec_ln_s, spec_ln_b,
                  spec_w1, spec_b1, spec_w2, spec_b2)
    out2 = _experts(xhat, y1, coef, gen_ln_s, gen_ln_b,
                    gen_w1, gen_b1, gen_w2, gen_b2)

    return out2[None], logits[None], probs[None], aw[None]


# speedup vs baseline: 1.9052x; 1.0792x over previous
import functools

import jax
import jax.numpy as jnp
from jax import lax
from jax.experimental import pallas as pl
from jax.experimental.pallas import tpu as pltpu
from jax.experimental.pallas import tpu_sc as plsc

S, D, H, E, DFF = 2048, 1024, 16, 8, 4096
DH = D // H
NEG = -1e30
BMG = 128
NPAD = 2 * S + E * BMG


def _ln_qkv_body(x_ref, s_ref, b_ref, w_ref, wb_ref, o_ref):
    xb = x_ref[...]
    m = jnp.mean(xb, axis=-1, keepdims=True)
    v = jnp.mean((xb - m) ** 2, axis=-1, keepdims=True)
    xn = (xb - m) * jax.lax.rsqrt(v + 1e-5) * s_ref[...] + b_ref[...]
    o_ref[...] = jax.lax.dot_general(
        xn, w_ref[...], (((1,), (1,)), ((), ())),
        preferred_element_type=jnp.float32) + wb_ref[...]


def _ln_qkv(x2, n1_s, n1_b, w, b):
    BM, BN = 256, 512
    return pl.pallas_call(
        _ln_qkv_body,
        grid=(S // BM, 3 * D // BN),
        in_specs=[
            pl.BlockSpec((BM, D), lambda i, j: (i, 0)),
            pl.BlockSpec((1, D), lambda i, j: (0, 0)),
            pl.BlockSpec((1, D), lambda i, j: (0, 0)),
            pl.BlockSpec((BN, D), lambda i, j: (j, 0)),
            pl.BlockSpec((1, BN), lambda i, j: (0, j)),
        ],
        out_specs=pl.BlockSpec((BM, BN), lambda i, j: (i, j)),
        out_shape=jax.ShapeDtypeStruct((S, 3 * D), jnp.float32),
    )(x2, n1_s.reshape(1, D), n1_b.reshape(1, D), w, b.reshape(1, 3 * D))


def _attn_body(q_ref, k_ref, v_ref, ao_ref, aw_ref):
    h = pl.program_id(1)
    q = q_ref[0]
    k = k_ref[0]
    v = v_ref[0]
    s = jax.lax.dot_general(q, k, (((1,), (1,)), ((), ())),
                            preferred_element_type=jnp.float32) * (1.0 / 8.0)
    mx = jnp.max(s, axis=-1, keepdims=True)
    p = jnp.exp(s - mx)
    p = p / jnp.sum(p, axis=-1, keepdims=True)
    ao_ref[0] = jax.lax.dot_general(p, v, (((1,), (0,)), ((), ())),
                                    preferred_element_type=jnp.float32)

    @pl.when(h == 0)
    def _():
        aw_ref[...] = p * (1.0 / H)

    @pl.when(h != 0)
    def _():
        aw_ref[...] += p * (1.0 / H)


def _attention(qkv):
    qkv3 = qkv.reshape(S, 3 * H, DH).transpose(1, 0, 2)
    BM = 512
    ao3, aw = pl.pallas_call(
        _attn_body,
        grid=(S // BM, H),
        in_specs=[
            pl.BlockSpec((1, BM, DH), lambda i, h: (h, i, 0)),
            pl.BlockSpec((1, S, DH), lambda i, h: (H + h, 0, 0)),
            pl.BlockSpec((1, S, DH), lambda i, h: (2 * H + h, 0, 0)),
        ],
        out_specs=[
            pl.BlockSpec((1, BM, DH), lambda i, h: (h, i, 0)),
            pl.BlockSpec((BM, S), lambda i, h: (i, 0)),
        ],
        out_shape=[
            jax.ShapeDtypeStruct((H, S, DH), jnp.float32),
            jax.ShapeDtypeStruct((S, S), jnp.float32),
        ],
    )(qkv3, qkv3, qkv3)
    return ao3.transpose(1, 0, 2).reshape(S, D), aw


def _post_attn_body(ao_ref, wo_ref, bo_ref, xr_ref, s2_ref, b2_ref,
                    rw_ref, rb_ref,
                    x1_ref, xhat_ref, logits_ref, probs_ref, i12_ref,
                    w12_ref):
    x1 = jax.lax.dot_general(ao_ref[...], wo_ref[...], (((1,), (1,)), ((), ())),
                             preferred_element_type=jnp.float32)
    x1 = x1 + bo_ref[...] + xr_ref[...]
    x1_ref[...] = x1
    m = jnp.mean(x1, axis=-1, keepdims=True)
    v = jnp.mean((x1 - m) ** 2, axis=-1, keepdims=True)
    xhat = (x1 - m) * jax.lax.rsqrt(v + 1e-5)
    xhat_ref[...] = xhat
    xn2 = xhat * s2_ref[...] + b2_ref[...]
    logits = jax.lax.dot_general(xn2, rw_ref[...], (((1,), (1,)), ((), ())),
                                 preferred_element_type=jnp.float32) + rb_ref[...]
    logits_ref[...] = logits
    mx = jnp.max(logits, axis=-1, keepdims=True)
    ex = jnp.exp(logits - mx)
    probs = ex / jnp.sum(ex, axis=-1, keepdims=True)
    probs_ref[...] = probs
    lane = jax.lax.broadcasted_iota(jnp.int32, probs.shape, 1)
    v1 = jnp.max(probs, axis=-1, keepdims=True)
    i1 = jnp.min(jnp.where(probs == v1, lane, E), axis=-1, keepdims=True)
    m1 = lane == i1
    pm = jnp.where(m1, NEG, probs)
    v2 = jnp.max(pm, axis=-1, keepdims=True)
    i2 = jnp.min(jnp.where(pm == v2, lane, E), axis=-1, keepdims=True)
    m2 = lane == i2
    b = jnp.exp(v2 - v1)
    w1 = 1.0 / (1.0 + b)
    w2 = b / (1.0 + b)
    i12_ref[...] = jnp.concatenate([i1, i2], axis=1)
    w12_ref[...] = jnp.concatenate([w1, w2], axis=1)


def _post_attn(ao, wo, bo, xres, n2_s, n2_b, rw, rb):
    BM = 256
    return pl.pallas_call(
        _post_attn_body,
        grid=(S // BM,),
        in_specs=[
            pl.BlockSpec((BM, D), lambda i: (i, 0)),
            pl.BlockSpec((D, D), lambda i: (0, 0)),
            pl.BlockSpec((1, D), lambda i: (0, 0)),
            pl.BlockSpec((BM, D), lambda i: (i, 0)),
            pl.BlockSpec((1, D), lambda i: (0, 0)),
            pl.BlockSpec((1, D), lambda i: (0, 0)),
            pl.BlockSpec((E, D), lambda i: (0, 0)),
            pl.BlockSpec((1, E), lambda i: (0, 0)),
        ],
        out_specs=[
            pl.BlockSpec((BM, D), lambda i: (i, 0)),
            pl.BlockSpec((BM, D), lambda i: (i, 0)),
            pl.BlockSpec((BM, E), lambda i: (i, 0)),
            pl.BlockSpec((BM, E), lambda i: (i, 0)),
            pl.BlockSpec((BM, 2), lambda i: (i, 0)),
            pl.BlockSpec((BM, 2), lambda i: (i, 0)),
        ],
        out_shape=[
            jax.ShapeDtypeStruct((S, D), jnp.float32),
            jax.ShapeDtypeStruct((S, D), jnp.float32),
            jax.ShapeDtypeStruct((S, E), jnp.float32),
            jax.ShapeDtypeStruct((S, E), jnp.float32),
            jax.ShapeDtypeStruct((S, 2), jnp.int32),
            jax.ShapeDtypeStruct((S, 2), jnp.float32),
        ],
    )(ao, wo, bo.reshape(1, D), xres, n2_s.reshape(1, D), n2_b.reshape(1, D),
      rw, rb.reshape(1, E))


def _gelu(x):
    return x * 0.5 * (1.0 + jax.lax.erf(x * (2.0 ** -0.5)))


def _experts_body(nE, xh_ref, acc_ref, sc_ref, ls_ref, lb_ref,
                  w1_ref, b1_ref, w2_ref, b2_ref, o_ref):
    e = pl.program_id(1)
    f = pl.program_id(2)

    @pl.when((e == 0) & (f == 0))
    def _():
        o_ref[...] = acc_ref[...]

    xin = xh_ref[...] * ls_ref[0] + lb_ref[0]
    h = _gelu(jax.lax.dot_general(xin, w1_ref[0], (((1,), (1,)), ((), ())),
                                  preferred_element_type=jnp.float32) + b1_ref[0])
    y = jax.lax.dot_general(h, w2_ref[0], (((1,), (1,)), ((), ())),
                            preferred_element_type=jnp.float32)
    lane = jax.lax.broadcasted_iota(jnp.int32, sc_ref.shape, 1)
    scale = jnp.sum(jnp.where(lane == e, sc_ref[...], 0.0), axis=-1,
                    keepdims=True)

    @pl.when(f == 0)
    def _():
        o_ref[...] += y * scale + scale * b2_ref[0]

    @pl.when(f != 0)
    def _():
        o_ref[...] += y * scale


def _experts(xhat, acc, scales, ln_s, ln_b, w1, b1, w2, b2):
    nE = w1.shape[0]
    BM, BF = 256, 1024
    return pl.pallas_call(
        functools.partial(_experts_body, nE),
        grid=(S // BM, nE, DFF // BF),
        in_specs=[
            pl.BlockSpec((BM, D), lambda s, e, f: (s, 0)),
            pl.BlockSpec((BM, D), lambda s, e, f: (s, 0)),
            pl.BlockSpec((BM, nE), lambda s, e, f: (s, 0)),
            pl.BlockSpec((1, 1, D), lambda s, e, f: (e, 0, 0)),
            pl.BlockSpec((1, 1, D), lambda s, e, f: (e, 0, 0)),
            pl.BlockSpec((1, BF, D), lambda s, e, f: (e, f, 0)),
            pl.BlockSpec((1, 1, BF), lambda s, e, f: (e, 0, f)),
            pl.BlockSpec((1, D, BF), lambda s, e, f: (e, 0, f)),
            pl.BlockSpec((1, 1, D), lambda s, e, f: (e, 0, 0)),
        ],
        out_specs=pl.BlockSpec((BM, D), lambda s, e, f: (s, 0)),
        out_shape=jax.ShapeDtypeStruct((S, D), jnp.float32),
    )(xhat, acc, scales, ln_s.reshape(nE, 1, D), ln_b.reshape(nE, 1, D),
      w1, b1.reshape(nE, 1, DFF), w2, b2.reshape(nE, 1, D))


def _route_metadata(i12, w12):
    ids = jnp.concatenate([i12[:, 0], i12[:, 1]])
    tok = jnp.tile(jnp.arange(S, dtype=jnp.int32), 2)
    wts = jnp.concatenate([w12[:, 0], w12[:, 1]])
    order = jnp.argsort(ids, stable=True)
    sids = ids[order]
    counts = jnp.bincount(ids, length=E).astype(jnp.int32)
    seg_start = jnp.concatenate(
        [jnp.zeros((1,), jnp.int32), jnp.cumsum(counts)])[:E]
    padded = ((counts + BMG - 1) // BMG) * BMG
    pstart = jnp.concatenate(
        [jnp.zeros((1,), jnp.int32), jnp.cumsum(padded)])[:E]
    r = jnp.arange(2 * S, dtype=jnp.int32)
    pos = pstart[sids] + (r - seg_start[sids])
    tok_p = jnp.zeros((NPAD,), jnp.int32).at[pos].set(tok[order])
    w_p = jnp.zeros((NPAD,), jnp.float32).at[pos].set(wts[order])
    bstart = jnp.arange(NPAD // BMG, dtype=jnp.int32) * BMG
    block_e = (jnp.sum(bstart[:, None] >= pstart[None, :], axis=1)
               .astype(jnp.int32) - 1)
    pos_orig = jnp.zeros((2 * S,), jnp.int32).at[order].set(pos)
    return tok_p, w_p, block_e, pos_orig


def _sc_gather(table, idx, nrows, chunk):
    info = plsc.get_sparse_core_info()
    nw = info.num_cores * info.num_subcores
    b_per_w = nrows // nw
    nchunks = b_per_w // chunk
    mesh = plsc.VectorSubcoreMesh(core_axis_name="c", subcore_axis_name="s")

    @functools.partial(
        pl.kernel, mesh=mesh,
        out_type=jax.ShapeDtypeStruct((nrows, D), jnp.float32),
        scratch_types=[
            pltpu.VMEM((chunk,), jnp.int32),
            pltpu.VMEM((chunk, D), jnp.float32),
            pltpu.SemaphoreType.DMA,
        ],
    )
    def k(table_hbm, idx_hbm, out_hbm, idx_v, rows_v, sem):
        wid = lax.axis_index("s") * info.num_cores + lax.axis_index("c")
        base = wid * b_per_w
        for c in range(nchunks):
            off = base + c * chunk
            pltpu.sync_copy(idx_hbm.at[pl.ds(off, chunk)], idx_v)
            pltpu.async_copy(table_hbm.at[idx_v], rows_v, sem).wait()
            pltpu.sync_copy(rows_v, out_hbm.at[pl.ds(off, chunk)])

    return k(table, idx)


def _grouped_body(be_ref, xg_ref, wrow_ref, ls_ref, lb_ref,
                  w1_ref, b1_ref, w2_ref, b2_ref, o_ref):
    f = pl.program_id(1)
    xin = xg_ref[...] * ls_ref[0] + lb_ref[0]
    h = _gelu(jax.lax.dot_general(xin, w1_ref[0], (((1,), (1,)), ((), ())),
                                  preferred_element_type=jnp.float32) + b1_ref[0])
    y = jax.lax.dot_general(h, w2_ref[0], (((1,), (1,)), ((), ())),
                            preferred_element_type=jnp.float32)
    w = wrow_ref[...]

    @pl.when(f == 0)
    def _():
        o_ref[...] = y * w + w * b2_ref[0]

    @pl.when(f != 0)
    def _():
        o_ref[...] += y * w


def _grouped_ffn(xg, w_p, block_e, ln_s, ln_b, w1, b1, w2, b2):
    BF = 1024
    nb = NPAD // BMG
    grid_spec = pltpu.PrefetchScalarGridSpec(
        num_scalar_prefetch=1,
        grid=(nb, DFF // BF),
        in_specs=[
            pl.BlockSpec((BMG, D), lambda i, f, be: (i, 0)),
            pl.BlockSpec((BMG, 1), lambda i, f, be: (i, 0)),
            pl.BlockSpec((1, 1, D), lambda i, f, be: (be[i], 0, 0)),
            pl.BlockSpec((1, 1, D), lambda i, f, be: (be[i], 0, 0)),
            pl.BlockSpec((1, BF, D), lambda i, f, be: (be[i], f, 0)),
            pl.BlockSpec((1, 1, BF), lambda i, f, be: (be[i], 0, f)),
            pl.BlockSpec((1, D, BF), lambda i, f, be: (be[i], 0, f)),
            pl.BlockSpec((1, 1, D), lambda i, f, be: (be[i], 0, 0)),
        ],
        out_specs=pl.BlockSpec((BMG, D), lambda i, f, be: (i, 0)),
    )
    return pl.pallas_call(
        _grouped_body,
        grid_spec=grid_spec,
        out_shape=jax.ShapeDtypeStruct((NPAD, D), jnp.float32),
    )(block_e, xg, w_p.reshape(NPAD, 1), ln_s.reshape(E, 1, D),
      ln_b.reshape(E, 1, D), w1, b1.reshape(E, 1, DFF), w2,
      b2.reshape(E, 1, D))


def _combine_body(a_ref, z0_ref, z1_ref, o_ref):
    o_ref[...] = a_ref[...] + z0_ref[...] + z1_ref[...]


def _combine(a, z):
    BM = 256
    return pl.pallas_call(
        _combine_body,
        grid=(S // BM,),
        in_specs=[
            pl.BlockSpec((BM, D), lambda i: (i, 0)),
            pl.BlockSpec((BM, D), lambda i: (i, 0)),
            pl.BlockSpec((BM, D), lambda i: (i + S // BM, 0)),
        ],
        out_specs=pl.BlockSpec((BM, D), lambda i: (i, 0)),
        out_shape=jax.ShapeDtypeStruct((S, D), jnp.float32),
    )(a, z, z)


def kernel(x, modality_type, n1_s, n1_b, n2_s, n2_b, attn_in_w, attn_in_b,
           attn_out_w, attn_out_b, router_w, router_b, spec_ln_s, spec_ln_b,
           spec_w1, spec_b1, spec_w2, spec_b2, gen_ln_s, gen_ln_b, gen_w1,
           gen_b1, gen_w2, gen_b2):
    x2 = x[0]
    mflat = modality_type[0]

    qkv = _ln_qkv(x2, n1_s, n1_b, attn_in_w, attn_in_b)
    ao, aw = _attention(qkv)
    x1, xhat, logits, probs, i12, w12 = _post_attn(
        ao, attn_out_w, attn_out_b, x2, n2_s, n2_b, router_w, router_b)

    tok_p, w_p, block_e, pos_orig = _route_metadata(i12, w12)
    xg = _sc_gather(xhat, tok_p, NPAD, 80)
    yg = _grouped_ffn(xg, w_p, block_e, gen_ln_s, gen_ln_b,
                      gen_w1, gen_b1, gen_w2, gen_b2)
    z = _sc_gather(yg, pos_orig, 2 * S, 64)

    ones = jnp.ones((S,), jnp.float32)
    scales_spec = jnp.stack(
        [ones, (mflat == 0).astype(jnp.float32),
         (mflat == 1).astype(jnp.float32)], axis=1)
    y1 = _experts(xhat, x1, scales_spec, spec_ln_s, spec_ln_b,
                  spec_w1, spec_b1, spec_w2, spec_b2)
    out2 = _combine(y1, z)

    return out2[None], logits[None], probs[None], aw[None]
